# Initial kernel scaffold; baseline (speedup 1.0000x reference)
import jax
import jax.numpy as jnp
from jax.experimental import pallas as pl


def kernel(x, edge_index, edge_attr, W_pre, b_pre, Wl0, bl0, Wr0, Wl1, bl1, Wr1, Wl2, bl2, Wr2, ln_w, ln_b):
    raise NotImplementedError("write your pallas kernel here")


# speedup vs baseline: 4.8956x; 4.8956x over previous
import functools

import jax
import jax.numpy as jnp
from jax import lax
from jax.experimental import pallas as pl
from jax.experimental.pallas import tpu as pltpu
from jax.experimental.pallas import tpu_sc as plsc

N = 10000
E = 320000
HID = 128
NC, NS = 2, 16
NW = NC * NS
NB = 79
PER = NB * 128
EPAD = NW * PER
RPT = 640
PADN = NS * RPT
BR = 400
NGRID = N // BR
NTOT = float(N * HID)


def _sc_segsum_body(hid_hbm, srcI, dstI, z2, z1, on, psum, pdeg,
                    src_v, dst_v, rows_v, ones_v, acc, dacc, gsem):
    c = lax.axis_index("c")
    s = lax.axis_index("s")
    wid = s * jnp.int32(NC) + c
    base = s * jnp.int32(RPT)
    pltpu.sync_copy(z2, acc.at[pl.ds(base, RPT)])
    pltpu.sync_copy(z1, dacc.at[pl.ds(base, RPT)])
    pltpu.sync_copy(on, ones_v)
    pltpu.sync_copy(srcI.at[wid], src_v)
    pltpu.sync_copy(dstI.at[wid], dst_v)
    plsc.subcore_barrier()

    @pl.loop(jnp.int32(0), jnp.int32(NB))
    def _(j):
        pltpu.async_copy(hid_hbm.at[src_v.at[j]], rows_v, gsem).wait()
        pltpu.sync_copy(rows_v, acc.at[dst_v.at[j]], add=True)
        pltpu.sync_copy(ones_v, dacc.at[dst_v.at[j]], add=True)

    plsc.subcore_barrier()
    pltpu.sync_copy(acc.at[pl.ds(base, RPT)], psum.at[c].at[pl.ds(base, RPT)])
    pltpu.sync_copy(dacc.at[pl.ds(base, RPT)], pdeg.at[c].at[pl.ds(base, RPT)])


_sc_segsum = functools.partial(
    pl.kernel,
    out_type=(
        jax.ShapeDtypeStruct((NC, PADN, HID), jnp.float32),
        jax.ShapeDtypeStruct((NC, PADN), jnp.float32),
    ),
    mesh=plsc.VectorSubcoreMesh(core_axis_name="c", subcore_axis_name="s",
                                num_cores=NC, num_subcores=NS),
    scratch_types=[
        pltpu.VMEM((NB, 128), jnp.int32),
        pltpu.VMEM((NB, 128), jnp.int32),
        pltpu.VMEM((128, HID), jnp.float32),
        pltpu.VMEM((128,), jnp.float32),
        pltpu.VMEM_SHARED((PADN, HID), jnp.float32),
        pltpu.VMEM_SHARED((PADN,), jnp.float32),
        pltpu.SemaphoreType.DMA,
    ],
)(_sc_segsum_body)


def _pre_body(x_ref, w_ref, b_ref, o_ref):
    o_ref[...] = jnp.maximum(
        jnp.dot(x_ref[...], w_ref[...], preferred_element_type=jnp.float32)
        + b_ref[...], 0.0)


def _combine_body(psum_ref, pdeg_ref, hid_ref, wl_ref, bl_ref, wr_ref,
                  h2_ref, stats_ref):
    i = pl.program_id(0)
    ssum = psum_ref[0] + psum_ref[1]
    deg = pdeg_ref[0] + pdeg_ref[1]
    agg = ssum * (1.0 / jnp.maximum(deg, 1.0))
    pre = (jnp.dot(agg, wl_ref[...], preferred_element_type=jnp.float32)
           + jnp.dot(hid_ref[...], wr_ref[...],
                     preferred_element_type=jnp.float32)
           + bl_ref[...])
    h2 = jnp.maximum(pre, 0.0)
    h2_ref[...] = h2
    lane = lax.broadcasted_iota(jnp.int32, (1, 128), 1)
    contrib = (jnp.where(lane == 0, jnp.sum(h2), 0.0)
               + jnp.where(lane == 1, jnp.sum(h2 * h2), 0.0))

    @pl.when(i == 0)
    def _():
        stats_ref[...] = jnp.zeros_like(stats_ref)

    stats_ref[...] += contrib


def _norm_body(h2_ref, stats_ref, w_ref, b_ref, o_ref):
    v = stats_ref[...]
    lane = lax.broadcasted_iota(jnp.int32, (1, 128), 1)
    tot = jnp.sum(jnp.where(lane == 0, v, 0.0))
    totq = jnp.sum(jnp.where(lane == 1, v, 0.0))
    mean = tot / NTOT
    var = totq / NTOT - mean * mean
    scale = lax.rsqrt(var + 1e-5)
    o_ref[...] = (h2_ref[...] - mean) * scale * w_ref[...] + b_ref[...]


def _pre_call(x, w, b):
    return pl.pallas_call(
        _pre_body,
        out_shape=jax.ShapeDtypeStruct((N, HID), jnp.float32),
    )(x, w, b)


def _combine_call(psum, pdeg3, hid, wl, bl, wr):
    return pl.pallas_call(
        _combine_body,
        grid=(NGRID,),
        in_specs=[
            pl.BlockSpec((NC, BR, HID), lambda i: (i * 0, i, i * 0)),
            pl.BlockSpec((NC, BR, 1), lambda i: (i * 0, i, i * 0)),
            pl.BlockSpec((BR, HID), lambda i: (i, i * 0)),
            pl.BlockSpec((HID, HID), lambda i: (i * 0, i * 0)),
            pl.BlockSpec((1, HID), lambda i: (i * 0, i * 0)),
            pl.BlockSpec((HID, HID), lambda i: (i * 0, i * 0)),
        ],
        out_specs=[
            pl.BlockSpec((BR, HID), lambda i: (i, i * 0)),
            pl.BlockSpec((1, 128), lambda i: (i * 0, i * 0)),
        ],
        out_shape=[
            jax.ShapeDtypeStruct((N, HID), jnp.float32),
            jax.ShapeDtypeStruct((1, 128), jnp.float32),
        ],
    )(psum, pdeg3, hid, wl, bl, wr)


def _norm_call(h2, stats, w2, b2):
    return pl.pallas_call(
        _norm_body,
        out_shape=jax.ShapeDtypeStruct((N, HID), jnp.float32),
    )(h2, stats, w2, b2)


def kernel(x, edge_index, edge_attr, W_pre, b_pre, Wl0, bl0, Wr0,
           Wl1, bl1, Wr1, Wl2, bl2, Wr2, ln_w, ln_b):
    del edge_attr
    ei = edge_index.astype(jnp.int32)
    srcI = jnp.pad(ei[0], (0, EPAD - E)).reshape(NW, NB, 128)
    dstI = jnp.pad(ei[1], (0, EPAD - E),
                   constant_values=N).reshape(NW, NB, 128)
    z2 = jnp.zeros((RPT, HID), jnp.float32)
    z1 = jnp.zeros((RPT,), jnp.float32)
    on = jnp.ones((128,), jnp.float32)

    x = x.astype(jnp.float32)
    b_pre2 = b_pre.reshape(1, HID).astype(jnp.float32)
    w2 = ln_w.reshape(1, HID).astype(jnp.float32)
    b2 = ln_b.reshape(1, HID).astype(jnp.float32)

    hidden = _pre_call(x, W_pre.astype(jnp.float32), b_pre2)
    for (wl, bl, wr) in ((Wl0, bl0, Wr0), (Wl1, bl1, Wr1), (Wl2, bl2, Wr2)):
        psum, pdeg = _sc_segsum(hidden, srcI, dstI, z2, z1, on)
        pdeg3 = pdeg.reshape(NC, PADN, 1)
        h2, stats = _combine_call(psum, pdeg3, hidden,
                                  wl.astype(jnp.float32),
                                  bl.reshape(1, HID).astype(jnp.float32),
                                  wr.astype(jnp.float32))
        hidden = _norm_call(h2, stats, w2, b2)
    return hidden
